# Initial kernel scaffold; baseline (speedup 1.0000x reference)
import jax
import jax.numpy as jnp
from jax.experimental import pallas as pl


def kernel(x, ln_scale, ln_bias, Wg, W1, b1, W2, b2):
    raise NotImplementedError("write your pallas kernel here")


# speedup vs baseline: 1.5849x; 1.5849x over previous
import functools

import jax
import jax.numpy as jnp
import numpy as np
from jax.experimental import pallas as pl
from jax.experimental.pallas import tpu as pltpu

B, S, D = 2, 2048, 768
E, K, P = 16, 2, 1536
T = B * S
TK = T * K
C = int(np.ceil(TK * 1.25 / E))
EC = E * C
TRASH = EC
TOK_BLK = 256


def _route_body(x_ref, scale_ref, bias_ref, wg_ref, tril_ref,
                xn_ref, dst_ref, wv_ref, cnt_ref):
    i = pl.program_id(0)

    @pl.when(i == 0)
    def _():
        cnt_ref[...] = jnp.zeros_like(cnt_ref)

    xb = x_ref[...]
    mu = jnp.mean(xb, axis=-1, keepdims=True)
    xc = xb - mu
    var = jnp.mean(xc * xc, axis=-1, keepdims=True)
    xn = xc * jax.lax.rsqrt(var + 1e-5) * scale_ref[...] + bias_ref[...]
    xn_ref[...] = xn

    logits = jnp.dot(xn, wg_ref[...], preferred_element_type=jnp.float32,
                     precision=jax.lax.Precision.DEFAULT)
    eidx = jax.lax.broadcasted_iota(jnp.int32, (TOK_BLK, E), 1)
    m1 = jnp.max(logits, axis=-1, keepdims=True)
    i1 = jnp.min(jnp.where(logits >= m1, eidx, E), axis=-1, keepdims=True)
    l2 = jnp.where(eidx == i1, -jnp.inf, logits)
    m2 = jnp.max(l2, axis=-1, keepdims=True)
    i2 = jnp.min(jnp.where(l2 >= m2, eidx, E), axis=-1, keepdims=True)
    dexp = jnp.exp(m2 - m1)
    denom = 1.0 + dexp
    w1 = 1.0 / denom
    w2 = dexp / denom

    o1 = (eidx == i1).astype(jnp.float32)
    o2 = (eidx == i2).astype(jnp.float32)
    h = o1 + o2
    s = jnp.dot(tril_ref[...], h, preferred_element_type=jnp.float32) \
        + cnt_ref[...]
    cnt_ref[...] = cnt_ref[...] + jnp.sum(h, axis=0, keepdims=True)
    r1 = jnp.sum(s * o1, axis=-1, keepdims=True)
    r2 = jnp.sum(s * o2, axis=-1, keepdims=True)
    v1 = r1 < C
    v2 = r2 < C
    d1 = jnp.where(v1, i1 * C + r1.astype(jnp.int32), TRASH)
    d2 = jnp.where(v2, i2 * C + r2.astype(jnp.int32), TRASH)
    dst_ref[...] = jnp.concatenate([d1, d2], axis=1)
    wv_ref[...] = jnp.concatenate([jnp.where(v1, w1, 0.0),
                                   jnp.where(v2, w2, 0.0)], axis=1)


@functools.partial(jax.jit, static_argnames=("interpret",))
def _route(xf, ln_scale, ln_bias, wg, interpret=False):
    tril = jnp.tril(jnp.ones((TOK_BLK, TOK_BLK), jnp.float32), -1)
    grid = (T // TOK_BLK,)
    return pl.pallas_call(
        _route_body,
        grid=grid,
        in_specs=[
            pl.BlockSpec((TOK_BLK, D), lambda i: (i, 0)),
            pl.BlockSpec((1, D), lambda i: (0, 0)),
            pl.BlockSpec((1, D), lambda i: (0, 0)),
            pl.BlockSpec((D, E), lambda i: (0, 0)),
            pl.BlockSpec((TOK_BLK, TOK_BLK), lambda i: (0, 0)),
        ],
        out_specs=[
            pl.BlockSpec((TOK_BLK, D), lambda i: (i, 0)),
            pl.BlockSpec((TOK_BLK, 2), lambda i: (i, 0)),
            pl.BlockSpec((TOK_BLK, 2), lambda i: (i, 0)),
        ],
        out_shape=[
            jax.ShapeDtypeStruct((T, D), jnp.float32),
            jax.ShapeDtypeStruct((T, 2), jnp.int32),
            jax.ShapeDtypeStruct((T, 2), jnp.float32),
        ],
        scratch_shapes=[pltpu.VMEM((1, E), jnp.float32)],
        compiler_params=pltpu.CompilerParams(
            dimension_semantics=("arbitrary",)),
        interpret=interpret,
    )(xf, ln_scale.reshape(1, D), ln_bias.reshape(1, D), wg, tril)


def kernel(x, ln_scale, ln_bias, Wg, W1, b1, W2, b2):
    xf = x.reshape(T, D)
    xn, dst, wv = _route(xf, ln_scale, ln_bias, Wg)

    dstf = dst.reshape(-1)
    wf = wv.reshape(-1)
    tok = jnp.arange(TK) // K
    disp = jnp.zeros((EC + 8, D), jnp.float32).at[dstf].set(xn[tok])
    de = disp[:EC].reshape(E, C, D)
    hh = jax.nn.gelu(jnp.einsum('ecd,edp->ecp', de, W1) + b1[:, None, :],
                     approximate=False)
    y = jnp.einsum('ecp,epd->ecd', hh, W2) + b2[:, None, :]
    yflat = jnp.concatenate([y.reshape(EC, D), jnp.zeros((8, D))], 0)
    contrib = yflat[dstf] * wf[:, None]
    moe = jnp.zeros((T, D), jnp.float32).at[tok].add(contrib)
    out = xf + moe
    return out.reshape(B, S, D)


# speedup vs baseline: 2.1411x; 1.3510x over previous
import functools

import jax
import jax.numpy as jnp
import numpy as np
from jax.experimental import pallas as pl
from jax.experimental.pallas import tpu as pltpu

B, S, D = 2, 2048, 768
E, K, P = 16, 2, 1536
T = B * S
TK = T * K
C = int(np.ceil(TK * 1.25 / E))
EC = E * C
TRASH = EC
TOK_BLK = 256


def _route_body(x_ref, scale_ref, bias_ref, wg_ref, tril_ref,
                xn_ref, dst_ref, wv_ref, cnt_ref):
    i = pl.program_id(0)

    @pl.when(i == 0)
    def _():
        cnt_ref[...] = jnp.zeros_like(cnt_ref)

    xb = x_ref[...]
    mu = jnp.mean(xb, axis=-1, keepdims=True)
    xc = xb - mu
    var = jnp.mean(xc * xc, axis=-1, keepdims=True)
    xn = xc * jax.lax.rsqrt(var + 1e-5) * scale_ref[...] + bias_ref[...]
    xn_ref[...] = xn

    logits = jnp.dot(xn, wg_ref[...], preferred_element_type=jnp.float32,
                     precision=jax.lax.Precision.DEFAULT)
    eidx = jax.lax.broadcasted_iota(jnp.int32, (TOK_BLK, E), 1)
    m1 = jnp.max(logits, axis=-1, keepdims=True)
    i1 = jnp.min(jnp.where(logits >= m1, eidx, E), axis=-1, keepdims=True)
    l2 = jnp.where(eidx == i1, -jnp.inf, logits)
    m2 = jnp.max(l2, axis=-1, keepdims=True)
    i2 = jnp.min(jnp.where(l2 >= m2, eidx, E), axis=-1, keepdims=True)
    dexp = jnp.exp(m2 - m1)
    denom = 1.0 + dexp
    w1 = 1.0 / denom
    w2 = dexp / denom

    o1 = (eidx == i1).astype(jnp.float32)
    o2 = (eidx == i2).astype(jnp.float32)
    h = o1 + o2
    s = jnp.dot(tril_ref[...], h, preferred_element_type=jnp.float32) \
        + cnt_ref[...]
    cnt_ref[...] = cnt_ref[...] + jnp.sum(h, axis=0, keepdims=True)
    r1 = jnp.sum(s * o1, axis=-1, keepdims=True)
    r2 = jnp.sum(s * o2, axis=-1, keepdims=True)
    v1 = r1 < C
    v2 = r2 < C
    d1 = jnp.where(v1, i1 * C + r1.astype(jnp.int32), TRASH)
    d2 = jnp.where(v2, i2 * C + r2.astype(jnp.int32), TRASH)
    dst_ref[...] = jnp.concatenate([d1, d2], axis=1)
    wv_ref[...] = jnp.concatenate([jnp.where(v1, w1, 0.0),
                                   jnp.where(v2, w2, 0.0)], axis=1)


@functools.partial(jax.jit, static_argnames=("interpret",))
def _route(xf, ln_scale, ln_bias, wg, interpret=False):
    tril = jnp.tril(jnp.ones((TOK_BLK, TOK_BLK), jnp.float32), -1)
    grid = (T // TOK_BLK,)
    return pl.pallas_call(
        _route_body,
        grid=grid,
        in_specs=[
            pl.BlockSpec((TOK_BLK, D), lambda i: (i, 0)),
            pl.BlockSpec((1, D), lambda i: (0, 0)),
            pl.BlockSpec((1, D), lambda i: (0, 0)),
            pl.BlockSpec((D, E), lambda i: (0, 0)),
            pl.BlockSpec((TOK_BLK, TOK_BLK), lambda i: (0, 0)),
        ],
        out_specs=[
            pl.BlockSpec((TOK_BLK, D), lambda i: (i, 0)),
            pl.BlockSpec((TOK_BLK, 2), lambda i: (i, 0)),
            pl.BlockSpec((TOK_BLK, 2), lambda i: (i, 0)),
        ],
        out_shape=[
            jax.ShapeDtypeStruct((T, D), jnp.float32),
            jax.ShapeDtypeStruct((T, 2), jnp.int32),
            jax.ShapeDtypeStruct((T, 2), jnp.float32),
        ],
        scratch_shapes=[pltpu.VMEM((1, E), jnp.float32)],
        compiler_params=pltpu.CompilerParams(
            dimension_semantics=("arbitrary",)),
        interpret=interpret,
    )(xf, ln_scale.reshape(1, D), ln_bias.reshape(1, D), wg, tril)


CB = 640


def _ffn_body(disp_ref, w1_ref, b1_ref, w2_ref, b2_ref, wrow_ref, y_ref):
    xb = disp_ref[...].astype(jnp.bfloat16)
    a = jnp.dot(xb, w1_ref[0].astype(jnp.bfloat16),
                preferred_element_type=jnp.float32) + b1_ref[0]
    hh = 0.5 * a * (1.0 + jax.lax.erf(a * np.float32(1.0 / np.sqrt(2.0))))
    y = jnp.dot(hh.astype(jnp.bfloat16), w2_ref[0].astype(jnp.bfloat16),
                preferred_element_type=jnp.float32) + b2_ref[0]
    y_ref[...] = y * wrow_ref[...][:, 0:1]


@functools.partial(jax.jit, static_argnames=("interpret",))
def _ffn(disp, W1, b1, W2, b2, wrow, interpret=False):
    nb = C // CB
    return pl.pallas_call(
        _ffn_body,
        grid=(E, nb),
        in_specs=[
            pl.BlockSpec((CB, D), lambda e, j: (e * (C // CB) + j, 0)),
            pl.BlockSpec((1, D, P), lambda e, j: (e, 0, 0)),
            pl.BlockSpec((1, 1, P), lambda e, j: (e, 0, 0)),
            pl.BlockSpec((1, P, D), lambda e, j: (e, 0, 0)),
            pl.BlockSpec((1, 1, D), lambda e, j: (e, 0, 0)),
            pl.BlockSpec((CB, 16), lambda e, j: (e * (C // CB) + j, 0)),
        ],
        out_specs=pl.BlockSpec((CB, D), lambda e, j: (e * (C // CB) + j, 0)),
        out_shape=jax.ShapeDtypeStruct((EC, D), jnp.float32),
        compiler_params=pltpu.CompilerParams(
            dimension_semantics=("arbitrary", "arbitrary")),
        interpret=interpret,
    )(disp, W1, b1.reshape(E, 1, P), W2, b2.reshape(E, 1, D), wrow)


def kernel(x, ln_scale, ln_bias, Wg, W1, b1, W2, b2):
    xf = x.reshape(T, D)
    xn, dst, wv = _route(xf, ln_scale, ln_bias, Wg)

    dstf = dst.reshape(-1)
    wf = wv.reshape(-1)
    tok = jnp.arange(TK) // K
    disp = jnp.zeros((EC + 8, D), jnp.float32).at[dstf].set(xn[tok])
    wrow = jnp.zeros((EC + 8, 16), jnp.float32).at[dstf, 0].set(wf)
    yw = _ffn(disp, W1, b1, W2, b2, wrow)
    yflat = jnp.concatenate([yw, jnp.zeros((8, D))], 0)
    moe = jnp.zeros((T, D), jnp.float32).at[tok].add(yflat[dstf])
    out = xf + moe
    return out.reshape(B, S, D)


# speedup vs baseline: 4.6189x; 2.1572x over previous
import functools

import jax
import jax.numpy as jnp
import numpy as np
from jax import lax
from jax.experimental import pallas as pl
from jax.experimental.pallas import tpu as pltpu
from jax.experimental.pallas import tpu_sc as plsc

B, S, D = 2, 2048, 768
E, K, P = 16, 2, 1536
T = B * S
TK = T * K
C = int(np.ceil(TK * 1.25 / E))
CP = C + 8
ECP = E * CP
TRASH = C
TOK_BLK = 256


def _route_body(x_ref, scale_ref, bias_ref, wg_ref, tril_ref,
                xn_ref, dst_ref, wv_ref, cnt_ref):
    i = pl.program_id(0)

    @pl.when(i == 0)
    def _():
        cnt_ref[...] = jnp.zeros_like(cnt_ref)

    xb = x_ref[...]
    mu = jnp.mean(xb, axis=-1, keepdims=True)
    xc = xb - mu
    var = jnp.mean(xc * xc, axis=-1, keepdims=True)
    xn = xc * jax.lax.rsqrt(var + 1e-5) * scale_ref[...] + bias_ref[...]
    xn_ref[...] = xn

    logits = jnp.dot(xn, wg_ref[...], preferred_element_type=jnp.float32,
                     precision=jax.lax.Precision.DEFAULT)
    eidx = jax.lax.broadcasted_iota(jnp.int32, (TOK_BLK, E), 1)
    m1 = jnp.max(logits, axis=-1, keepdims=True)
    i1 = jnp.min(jnp.where(logits >= m1, eidx, E), axis=-1, keepdims=True)
    l2 = jnp.where(eidx == i1, -jnp.inf, logits)
    m2 = jnp.max(l2, axis=-1, keepdims=True)
    i2 = jnp.min(jnp.where(l2 >= m2, eidx, E), axis=-1, keepdims=True)
    dexp = jnp.exp(m2 - m1)
    denom = 1.0 + dexp
    w1 = 1.0 / denom
    w2 = dexp / denom

    o1 = (eidx == i1).astype(jnp.float32)
    o2 = (eidx == i2).astype(jnp.float32)
    h = o1 + o2
    s = jnp.dot(tril_ref[...], h, preferred_element_type=jnp.float32) \
        + cnt_ref[...]
    cnt_ref[...] = cnt_ref[...] + jnp.sum(h, axis=0, keepdims=True)
    r1 = jnp.sum(s * o1, axis=-1, keepdims=True)
    r2 = jnp.sum(s * o2, axis=-1, keepdims=True)
    v1 = r1 < C
    v2 = r2 < C
    d1 = jnp.where(v1, i1 * CP + r1.astype(jnp.int32), TRASH)
    d2 = jnp.where(v2, i2 * CP + r2.astype(jnp.int32), TRASH)
    dst_ref[...] = jnp.concatenate([d1, d2], axis=1)
    w1m = jnp.where(v1, w1, 0.0)
    w2m = jnp.where(v2, w2, 0.0)
    wv_ref[...] = jnp.concatenate(
        [jnp.broadcast_to(w1m[:, None, :], (TOK_BLK, 1, 16)),
         jnp.broadcast_to(w2m[:, None, :], (TOK_BLK, 1, 16))], axis=1)


@functools.partial(jax.jit, static_argnames=("interpret",))
def _route(xf, ln_scale, ln_bias, wg, interpret=False):
    tril = jnp.tril(jnp.ones((TOK_BLK, TOK_BLK), jnp.float32), -1)
    grid = (T // TOK_BLK,)
    return pl.pallas_call(
        _route_body,
        grid=grid,
        in_specs=[
            pl.BlockSpec((TOK_BLK, D), lambda i: (i, 0)),
            pl.BlockSpec((1, D), lambda i: (0, 0)),
            pl.BlockSpec((1, D), lambda i: (0, 0)),
            pl.BlockSpec((D, E), lambda i: (0, 0)),
            pl.BlockSpec((TOK_BLK, TOK_BLK), lambda i: (0, 0)),
        ],
        out_specs=[
            pl.BlockSpec((TOK_BLK, D), lambda i: (i, 0)),
            pl.BlockSpec((TOK_BLK, 2), lambda i: (i, 0)),
            pl.BlockSpec((TOK_BLK, 2, 16), lambda i: (i, 0, 0)),
        ],
        out_shape=[
            jax.ShapeDtypeStruct((T, D), jnp.float32),
            jax.ShapeDtypeStruct((T, 2), jnp.int32),
            jax.ShapeDtypeStruct((T, 2, 16), jnp.float32),
        ],
        scratch_shapes=[pltpu.VMEM((1, E), jnp.float32)],
        compiler_params=pltpu.CompilerParams(
            dimension_semantics=("arbitrary",)),
        interpret=interpret,
    )(xf, ln_scale.reshape(1, D), ln_bias.reshape(1, D), wg, tril)


NC, NS, L = 2, 16, 16
NW = NC * NS
PAIRS_W = TK // NW
PCH = 64
TOK_W = T // NW
TCH = 32

_sc_mesh = plsc.VectorSubcoreMesh(core_axis_name="c", subcore_axis_name="s")


def _dispatch_body(xn_hbm, dst_hbm, disp_hbm, dstv, sidx, buf, sem):
    wid = lax.axis_index("s") * NC + lax.axis_index("c")

    def chunk(ci, carry):
        base = wid * PAIRS_W + ci * PCH
        pltpu.sync_copy(dst_hbm.at[pl.ds(base, PCH)], dstv)
        for g in range(PCH // L):
            pairv = base + g * L + lax.iota(jnp.int32, L)
            sidx[pl.ds(g * L, L)] = lax.shift_right_logical(pairv, 1)
        pltpu.async_copy(xn_hbm.at[sidx], buf, sem).wait()
        pltpu.async_copy(buf, disp_hbm.at[dstv], sem).wait()
        return carry

    lax.fori_loop(0, PAIRS_W // PCH, chunk, 0)


def _dispatch(xn, dstf):
    return pl.kernel(
        _dispatch_body,
        out_type=jax.ShapeDtypeStruct((ECP, D), jnp.float32),
        mesh=_sc_mesh,
        scratch_types=[
            pltpu.VMEM((PCH,), jnp.int32),
            pltpu.VMEM((PCH,), jnp.int32),
            pltpu.VMEM((PCH, D), jnp.float32),
            pltpu.SemaphoreType.DMA,
        ],
    )(xn, dstf)


def _combine_body(x_hbm, dst_hbm, wb_hbm, yw_hbm, out_hbm,
                  dstv, xbuf, wbuf, gbuf, sem):
    wid = lax.axis_index("s") * NC + lax.axis_index("c")

    def chunk(ci, carry):
        tb = wid * TOK_W + ci * TCH
        pltpu.sync_copy(x_hbm.at[pl.ds(tb, TCH)], xbuf)
        pltpu.sync_copy(dst_hbm.at[pl.ds(2 * tb, 2 * TCH)], dstv)
        pltpu.sync_copy(wb_hbm.at[pl.ds(2 * tb, 2 * TCH)], wbuf)
        pltpu.async_copy(yw_hbm.at[dstv], gbuf, sem).wait()

        def tok_row(t, c2):
            w1v = wbuf[2 * t, :]
            w2v = wbuf[2 * t + 1, :]

            def col(c, c3):
                sl = pl.ds(c * L, L)
                xbuf[t, sl] = (xbuf[t, sl] + w1v * gbuf[2 * t, sl]
                               + w2v * gbuf[2 * t + 1, sl])
                return c3
            lax.fori_loop(0, D // L, col, 0, unroll=4)
            return c2

        lax.fori_loop(0, TCH, tok_row, 0)
        pltpu.sync_copy(xbuf, out_hbm.at[pl.ds(tb, TCH)])
        return carry

    lax.fori_loop(0, TOK_W // TCH, chunk, 0)


def _combine(xf, dstf, wb, yw):
    return pl.kernel(
        _combine_body,
        out_type=jax.ShapeDtypeStruct((T, D), jnp.float32),
        mesh=_sc_mesh,
        scratch_types=[
            pltpu.VMEM((2 * TCH,), jnp.int32),
            pltpu.VMEM((TCH, D), jnp.float32),
            pltpu.VMEM((2 * TCH, 16), jnp.float32),
            pltpu.VMEM((2 * TCH, D), jnp.float32),
            pltpu.SemaphoreType.DMA,
        ],
    )(xf, dstf, wb, yw)


CB = CP


def _ffn_body(disp_ref, w1_ref, b1_ref, w2_ref, b2_ref, y_ref):
    xb = disp_ref[...].astype(jnp.bfloat16)
    a = jnp.dot(xb, w1_ref[0].astype(jnp.bfloat16),
                preferred_element_type=jnp.float32) + b1_ref[0]
    hh = 0.5 * a * (1.0 + jax.lax.erf(a * np.float32(1.0 / np.sqrt(2.0))))
    y_ref[...] = jnp.dot(hh.astype(jnp.bfloat16), w2_ref[0].astype(jnp.bfloat16),
                         preferred_element_type=jnp.float32) + b2_ref[0]


@functools.partial(jax.jit, static_argnames=("interpret",))
def _ffn(disp, W1, b1, W2, b2, interpret=False):
    return pl.pallas_call(
        _ffn_body,
        grid=(E,),
        in_specs=[
            pl.BlockSpec((CB, D), lambda e: (e, 0)),
            pl.BlockSpec((1, D, P), lambda e: (e, 0, 0)),
            pl.BlockSpec((1, 1, P), lambda e: (e, 0, 0)),
            pl.BlockSpec((1, P, D), lambda e: (e, 0, 0)),
            pl.BlockSpec((1, 1, D), lambda e: (e, 0, 0)),
        ],
        out_specs=pl.BlockSpec((CB, D), lambda e: (e, 0)),
        out_shape=jax.ShapeDtypeStruct((ECP, D), jnp.float32),
        compiler_params=pltpu.CompilerParams(
            dimension_semantics=("arbitrary",)),
        interpret=interpret,
    )(disp, W1, b1.reshape(E, 1, P), W2, b2.reshape(E, 1, D))


def kernel(x, ln_scale, ln_bias, Wg, W1, b1, W2, b2):
    xf = x.reshape(T, D)
    xn, dst, wv = _route(xf, ln_scale, ln_bias, Wg)
    dstf = dst.reshape(TK)
    wb = wv.reshape(TK, 16)
    disp = _dispatch(xn, dstf)
    yw = _ffn(disp, W1, b1, W2, b2)
    out = _combine(xf, dstf, wb, yw)
    return out.reshape(B, S, D)
